# baseline (speedup 1.0000x reference)
import functools
import jax
import jax.numpy as jnp
from jax import lax
from jax.experimental import pallas as pl
from jax.experimental.pallas import tpu as pltpu
from jax.experimental.pallas import tpu_sc as plsc

NC, NS = 2, 16
NW = NC * NS
R = 64
C = 16


def sc_add(x2, pos):
    RB, D = x2.shape
    S = pos.shape[0]
    B = RB // S
    rows_per_w = RB // NW
    n_chunks = rows_per_w // R
    mesh = plsc.VectorSubcoreMesh(
        core_axis_name="c", subcore_axis_name="s", num_cores=NC, num_subcores=NS
    )

    @functools.partial(
        pl.kernel,
        out_type=jax.ShapeDtypeStruct((RB, D), jnp.float32),
        mesh=mesh,
        scratch_types=[
            pltpu.VMEM((R, D), jnp.float32),
            pltpu.VMEM((C, D), jnp.float32),
        ],
    )
    def k(x_hbm, pos_hbm, out_hbm, xbuf, pbuf):
        cid = lax.axis_index("c")
        sid = lax.axis_index("s")
        wid = sid * NC + cid
        base = wid * rows_per_w

        def chunk_body(kk, carry):
            r0 = pl.multiple_of(base + kk * R, R)
            s0 = pl.multiple_of(r0 // B, C)
            pltpu.sync_copy(x_hbm.at[pl.ds(r0, R)], xbuf)
            pltpu.sync_copy(pos_hbm.at[pl.ds(s0, C)], pbuf)

            def c_body(c, inner):
                for g in range(D // 16):
                    sl = pl.ds(g * 16, 16)
                    p = pbuf[c, sl]
                    for b in range(B):
                        r = c * B + b
                        xbuf[r, sl] = xbuf[r, sl] + p
                return inner

            lax.fori_loop(0, C, c_body, 0)
            pltpu.sync_copy(xbuf, out_hbm.at[pl.ds(r0, R)])
            return carry

        lax.fori_loop(0, n_chunks, chunk_body, 0)

    return k(x2, pos)


def kernel(x, pos_embed):
    S, B, D = x.shape
    x2 = x.reshape(S * B, D)
    out2 = sc_add(x2, pos_embed[:S])
    return out2.reshape(S, B, D)


# speedup vs baseline: 1.7122x; 1.7122x over previous
import functools
import jax
import jax.numpy as jnp
from jax import lax
from jax.experimental import pallas as pl
from jax.experimental.pallas import tpu as pltpu
from jax.experimental.pallas import tpu_sc as plsc

NC, NS = 2, 16
NW = NC * NS
C = 16
SC_S = 1024
_BLK = 1024


def _sc_add(x, pos, s_count):
    S, B, D = x.shape
    s_per_w = s_count // NW
    n_chunks = s_per_w // C
    mesh = plsc.VectorSubcoreMesh(
        core_axis_name="c", subcore_axis_name="s", num_cores=NC, num_subcores=NS
    )

    @functools.partial(
        pl.kernel,
        out_type=jax.ShapeDtypeStruct((S, B, D), jnp.float32),
        mesh=mesh,
        scratch_types=[
            pltpu.VMEM((C, B, D), jnp.float32),
            pltpu.VMEM((C, D), jnp.float32),
        ],
    )
    def k(x_hbm, pos_hbm, out_hbm, xbuf, pbuf):
        wid = lax.axis_index("s") * NC + lax.axis_index("c")
        base = wid * s_per_w

        def chunk_body(kk, carry):
            s0 = pl.multiple_of(base + kk * C, C)
            pltpu.sync_copy(x_hbm.at[pl.ds(s0, C)], xbuf)
            pltpu.sync_copy(pos_hbm.at[pl.ds(s0, C)], pbuf)

            def c_body(c, inner):
                for g in range(D // 16):
                    sl = pl.ds(g * 16, 16)
                    p = pbuf[c, sl]
                    for b in range(B):
                        xbuf[c, b, sl] = xbuf[c, b, sl] + p
                return inner

            lax.fori_loop(0, C, c_body, 0)
            pltpu.sync_copy(xbuf, out_hbm.at[pl.ds(s0, C)])
            return carry

        lax.fori_loop(0, n_chunks, chunk_body, 0)

    return k(x, pos)


def _tc_kernel(x_ref, p_ref, o_ref):
    o_ref[...] = x_ref[...] + p_ref[...][:, None, :]


def _tc_add(x, pos, s_begin):
    S, B, D = x.shape
    off = s_begin // _BLK
    return pl.pallas_call(
        _tc_kernel,
        grid=((S - s_begin) // _BLK,),
        in_specs=[
            pl.BlockSpec((_BLK, B, D), lambda i: (i + off, 0, 0)),
            pl.BlockSpec((_BLK, D), lambda i: (i + off, 0)),
        ],
        out_specs=pl.BlockSpec((_BLK, B, D), lambda i: (i + off, 0, 0)),
        out_shape=jax.ShapeDtypeStruct((S, B, D), x.dtype),
    )(x, pos)


def kernel(x, pos_embed):
    S, B, D = x.shape
    pos = pos_embed[:S]
    sc_out = _sc_add(x, pos, SC_S)
    tc_out = _tc_add(x, pos, SC_S)
    return jnp.concatenate([sc_out[:SC_S], tc_out[SC_S:]], axis=0)


# speedup vs baseline: 5.6471x; 3.2981x over previous
import jax
import jax.numpy as jnp
from jax.experimental import pallas as pl


_BLK = 1024


def _add_kernel(x_ref, p_ref, o_ref):
    o_ref[...] = x_ref[...] + p_ref[...][:, None, :]


def kernel(x, pos_embed):
    S, B, D = x.shape
    blk = _BLK if S % _BLK == 0 else S
    return pl.pallas_call(
        _add_kernel,
        grid=(S // blk,),
        in_specs=[
            pl.BlockSpec((blk, B, D), lambda i: (i, 0, 0)),
            pl.BlockSpec((blk, D), lambda i: (i, 0)),
        ],
        out_specs=pl.BlockSpec((blk, B, D), lambda i: (i, 0, 0)),
        out_shape=jax.ShapeDtypeStruct((S, B, D), x.dtype),
    )(x, pos_embed[:S])
